# Initial kernel scaffold; baseline (speedup 1.0000x reference)
import jax
import jax.numpy as jnp
from jax.experimental import pallas as pl


def kernel(node_feature, edge_feature, edge_index, W_ni, W_nj, W_fij, b_egat, attn, W_node_src, W_edge_dense, b_edge_dense, W_node_dense, b_node_dense, ln_gamma, ln_beta):
    raise NotImplementedError("write your pallas kernel here")


# speedup vs baseline: 1.0386x; 1.0386x over previous
import functools
import jax
import jax.numpy as jnp
from jax import lax
from jax.experimental import pallas as pl
from jax.experimental.pallas import tpu as pltpu

_N = 10000
_E = 160000
_D = 256
_H = 4
_OF = 128
_HD = 512


def _mm_bias_kernel(x_ref, w_ref, b_ref, o_ref):
    o_ref[...] = jnp.dot(x_ref[...], w_ref[...],
                         preferred_element_type=jnp.float32) + b_ref[...]


def _mm_bias(x, w, b, block_rows):
    m, k = x.shape
    n = w.shape[1]
    grid = (m // block_rows,)
    return pl.pallas_call(
        _mm_bias_kernel,
        grid=grid,
        in_specs=[
            pl.BlockSpec((block_rows, k), lambda i: (i, 0)),
            pl.BlockSpec((k, n), lambda i: (0, 0)),
            pl.BlockSpec((1, n), lambda i: (0, 0)),
        ],
        out_specs=pl.BlockSpec((block_rows, n), lambda i: (i, 0)),
        out_shape=jax.ShapeDtypeStruct((m, n), jnp.float32),
    )(x, w, b.reshape(1, n))


def _edge_phase_kernel(fsum_ref, attn_ref, fout_ref, e_ref):
    f = fsum_ref[...]
    f = jnp.where(f > 0, f, 0.01 * f)
    fout_ref[...] = f
    a = attn_ref[...]
    fa = f * a
    r, _ = fa.shape
    e_ref[...] = jnp.sum(fa.reshape(r, _H, _OF), axis=-1)


def _edge_phase(f_sum, attn_flat, block_rows):
    e_rows = f_sum.shape[0]
    grid = (e_rows // block_rows,)
    return pl.pallas_call(
        _edge_phase_kernel,
        grid=grid,
        in_specs=[
            pl.BlockSpec((block_rows, _HD), lambda i: (i, 0)),
            pl.BlockSpec((1, _HD), lambda i: (0, 0)),
        ],
        out_specs=[
            pl.BlockSpec((block_rows, _HD), lambda i: (i, 0)),
            pl.BlockSpec((block_rows, _H), lambda i: (i, 0)),
        ],
        out_shape=[
            jax.ShapeDtypeStruct((e_rows, _HD), jnp.float32),
            jax.ShapeDtypeStruct((e_rows, _H), jnp.float32),
        ],
    )(f_sum, attn_flat.reshape(1, _HD))


def _dense_ln_kernel(x_ref, w_ref, b_ref, g_ref, bb_ref, o_ref):
    y = jnp.dot(x_ref[...], w_ref[...], preferred_element_type=jnp.float32)
    y = jnp.maximum(y + b_ref[...], 0.0)
    mu = jnp.mean(y, axis=-1, keepdims=True)
    var = jnp.mean((y - mu) ** 2, axis=-1, keepdims=True)
    o_ref[...] = (y - mu) * lax.rsqrt(var + 1e-5) * g_ref[...] + bb_ref[...]


def _dense_ln(x, w, b, g, beta, block_rows):
    m, k = x.shape
    n = w.shape[1]
    grid = (m // block_rows,)
    return pl.pallas_call(
        _dense_ln_kernel,
        grid=grid,
        in_specs=[
            pl.BlockSpec((block_rows, k), lambda i: (i, 0)),
            pl.BlockSpec((k, n), lambda i: (0, 0)),
            pl.BlockSpec((1, n), lambda i: (0, 0)),
            pl.BlockSpec((1, n), lambda i: (0, 0)),
            pl.BlockSpec((1, n), lambda i: (0, 0)),
        ],
        out_specs=pl.BlockSpec((block_rows, n), lambda i: (i, 0)),
        out_shape=jax.ShapeDtypeStruct((m, n), jnp.float32),
    )(x, w, b.reshape(1, n), g.reshape(1, n), beta.reshape(1, n))


def _node_div_dense_ln_kernel(acc_ref, den_ref, w_ref, b_ref, g_ref, bb_ref,
                              o_ref):
    acc = acc_ref[...]
    den = den_ref[...]
    den_w = jnp.repeat(den, _OF, axis=1)
    h = jnp.where(den_w > 0, acc / den_w, 0.0)
    y = jnp.dot(h, w_ref[...], preferred_element_type=jnp.float32)
    y = jnp.maximum(y + b_ref[...], 0.0)
    mu = jnp.mean(y, axis=-1, keepdims=True)
    var = jnp.mean((y - mu) ** 2, axis=-1, keepdims=True)
    o_ref[...] = (y - mu) * lax.rsqrt(var + 1e-5) * g_ref[...] + bb_ref[...]


def _node_div_dense_ln(acc, den, w, b, g, beta, block_rows):
    m = acc.shape[0]
    n = w.shape[1]
    grid = (m // block_rows,)
    return pl.pallas_call(
        _node_div_dense_ln_kernel,
        grid=grid,
        in_specs=[
            pl.BlockSpec((block_rows, _HD), lambda i: (i, 0)),
            pl.BlockSpec((block_rows, _H), lambda i: (i, 0)),
            pl.BlockSpec((_HD, n), lambda i: (0, 0)),
            pl.BlockSpec((1, n), lambda i: (0, 0)),
            pl.BlockSpec((1, n), lambda i: (0, 0)),
            pl.BlockSpec((1, n), lambda i: (0, 0)),
        ],
        out_specs=pl.BlockSpec((block_rows, n), lambda i: (i, 0)),
        out_shape=jax.ShapeDtypeStruct((m, n), jnp.float32),
    )(acc, den, w, b.reshape(1, n), g.reshape(1, n), beta.reshape(1, n))


def kernel(node_feature, edge_feature, edge_index, W_ni, W_nj, W_fij, b_egat,
           attn, W_node_src, W_edge_dense, b_edge_dense, W_node_dense,
           b_node_dense, ln_gamma, ln_beta):
    src = edge_index[0]
    dst = edge_index[1]

    w3 = jnp.concatenate([W_ni, W_nj, W_node_src], axis=1)
    b3 = jnp.zeros((3 * _HD,), jnp.float32)
    proj = _mm_bias(node_feature, w3, b3, block_rows=1000)
    f_ni = proj[:, :_HD]
    f_nj = proj[:, _HD:2 * _HD]
    h_src = proj[:, 2 * _HD:]

    f_fij = _mm_bias(edge_feature, W_fij, b_egat, block_rows=2000)

    f_sum = f_ni[src] + f_nj[dst] + f_fij

    f_out, e = _edge_phase(f_sum, attn.reshape(_HD), block_rows=2000)

    kmax = jnp.max(e)
    ee = jnp.exp(e - kmax)

    msg = h_src[src].reshape(_E, _H, _OF) * ee[:, :, None]
    acc = jax.ops.segment_sum(msg, dst, num_segments=_N).reshape(_N, _HD)
    den = jax.ops.segment_sum(ee, dst, num_segments=_N)

    edge_out = _dense_ln(f_out, W_edge_dense, b_edge_dense, ln_gamma, ln_beta,
                         block_rows=2000)
    node_out = _node_div_dense_ln(acc, den, W_node_dense, b_node_dense,
                                  ln_gamma, ln_beta, block_rows=1000)
    return (node_out, edge_out)


# speedup vs baseline: 1.0935x; 1.0528x over previous
import functools
import jax
import jax.numpy as jnp
from jax import lax
from jax.experimental import pallas as pl
from jax.experimental.pallas import tpu as pltpu
from jax.experimental.pallas import tpu_sc as plsc

_N = 10000
_E = 160000
_D = 256
_H = 4
_OF = 128
_HD = 512

_NW = 32
_EW = _E // _NW
_C = 40
_NCHUNK = _EW // _C


def _sc_mesh():
    return plsc.VectorSubcoreMesh(core_axis_name="c", subcore_axis_name="s",
                                  num_cores=2, num_subcores=16)


def _sc_gather_add_body(fni_hbm, fnj_hbm, src_hbm, dst_hbm, g_hbm,
                        src_v, dst_v, bufa, bufb, sema, semb):
    cid = lax.axis_index("c")
    sid = lax.axis_index("s")
    wid = sid * 2 + cid
    pltpu.sync_copy(src_hbm.at[wid], src_v)
    pltpu.sync_copy(dst_hbm.at[wid], dst_v)

    def chunk_body(c, carry):
        cpa = pltpu.async_copy(fni_hbm.at[src_v.at[c]], bufa, sema)
        cpb = pltpu.async_copy(fnj_hbm.at[dst_v.at[c]], bufb, semb)
        cpa.wait()
        cpb.wait()

        def row_body(r, carry2):
            def lane_body(l, carry3):
                sl = pl.ds(l * 16, 16)
                bufa[r, sl] = bufa[r, sl] + bufb[r, sl]
                return carry3
            return lax.fori_loop(0, _HD // 16, lane_body, carry2)
        lax.fori_loop(0, _C, row_body, carry)

        pltpu.sync_copy(bufa, g_hbm.at[pl.ds(wid * _EW + c * _C, _C)])
        return carry

    lax.fori_loop(0, _NCHUNK, chunk_body, 0)


def _sc_gather_add(f_ni, f_nj, src, dst):
    src2d = src.reshape(_NW, _NCHUNK, _C)
    dst2d = dst.reshape(_NW, _NCHUNK, _C)
    fn = pl.kernel(
        _sc_gather_add_body,
        out_type=jax.ShapeDtypeStruct((_E, _HD), jnp.float32),
        mesh=_sc_mesh(),
        scratch_types=[
            pltpu.VMEM((_NCHUNK, _C), jnp.int32),
            pltpu.VMEM((_NCHUNK, _C), jnp.int32),
            pltpu.VMEM((_C, _HD), jnp.float32),
            pltpu.VMEM((_C, _HD), jnp.float32),
            pltpu.SemaphoreType.DMA,
            pltpu.SemaphoreType.DMA,
        ],
    )
    return fn(f_ni, f_nj, src2d, dst2d)


def _proj_kernel(x_ref, w_ref, o1_ref, o2_ref, o3_ref):
    y = jnp.dot(x_ref[...], w_ref[...], preferred_element_type=jnp.float32)
    o1_ref[...] = y[:, :_HD]
    o2_ref[...] = y[:, _HD:2 * _HD]
    o3_ref[...] = y[:, 2 * _HD:]


def _proj(x, w3, block_rows):
    m, k = x.shape
    grid = (m // block_rows,)
    out = jax.ShapeDtypeStruct((m, _HD), jnp.float32)
    return pl.pallas_call(
        _proj_kernel,
        grid=grid,
        in_specs=[
            pl.BlockSpec((block_rows, k), lambda i: (i, 0)),
            pl.BlockSpec((k, 3 * _HD), lambda i: (0, 0)),
        ],
        out_specs=[pl.BlockSpec((block_rows, _HD), lambda i: (i, 0))] * 3,
        out_shape=[out, out, out],
    )(x, w3)


def _edge_phase_kernel(ef_ref, w_ref, b_ref, g_ref, attn_ref, fout_ref, e_ref):
    y = jnp.dot(ef_ref[...], w_ref[...], preferred_element_type=jnp.float32)
    f = y + b_ref[...] + g_ref[...]
    f = jnp.where(f > 0, f, 0.01 * f)
    fout_ref[...] = f
    fa = f * attn_ref[...]
    r = fa.shape[0]
    e_ref[...] = jnp.sum(fa.reshape(r, _H, _OF), axis=-1)


def _edge_phase(edge_feature, w_fij, b_egat, g, attn_flat, block_rows):
    grid = (_E // block_rows,)
    return pl.pallas_call(
        _edge_phase_kernel,
        grid=grid,
        in_specs=[
            pl.BlockSpec((block_rows, _D), lambda i: (i, 0)),
            pl.BlockSpec((_D, _HD), lambda i: (0, 0)),
            pl.BlockSpec((1, _HD), lambda i: (0, 0)),
            pl.BlockSpec((block_rows, _HD), lambda i: (i, 0)),
            pl.BlockSpec((1, _HD), lambda i: (0, 0)),
        ],
        out_specs=[
            pl.BlockSpec((block_rows, _HD), lambda i: (i, 0)),
            pl.BlockSpec((block_rows, _H), lambda i: (i, 0)),
        ],
        out_shape=[
            jax.ShapeDtypeStruct((_E, _HD), jnp.float32),
            jax.ShapeDtypeStruct((_E, _H), jnp.float32),
        ],
    )(edge_feature, w_fij, b_egat.reshape(1, _HD), g,
      attn_flat.reshape(1, _HD))


def _exp_kernel(e_ref, k_ref, o_ref):
    o_ref[...] = jnp.exp(e_ref[...] - k_ref[...])


def _exp_shift(e_flat, kmax, block_rows):
    m = e_flat.shape[0]
    grid = (m // block_rows,)
    return pl.pallas_call(
        _exp_kernel,
        grid=grid,
        in_specs=[
            pl.BlockSpec((block_rows, 128), lambda i: (i, 0)),
            pl.BlockSpec((1, 1), lambda i: (0, 0)),
        ],
        out_specs=pl.BlockSpec((block_rows, 128), lambda i: (i, 0)),
        out_shape=jax.ShapeDtypeStruct((m, 128), jnp.float32),
    )(e_flat, kmax.reshape(1, 1))


def _dense_ln_kernel(x_ref, w_ref, b_ref, g_ref, bb_ref, o_ref):
    y = jnp.dot(x_ref[...], w_ref[...], preferred_element_type=jnp.float32)
    y = jnp.maximum(y + b_ref[...], 0.0)
    mu = jnp.mean(y, axis=-1, keepdims=True)
    var = jnp.mean((y - mu) ** 2, axis=-1, keepdims=True)
    o_ref[...] = (y - mu) * lax.rsqrt(var + 1e-5) * g_ref[...] + bb_ref[...]


def _dense_ln(x, w, b, g, beta, block_rows):
    m, k = x.shape
    n = w.shape[1]
    grid = (m // block_rows,)
    return pl.pallas_call(
        _dense_ln_kernel,
        grid=grid,
        in_specs=[
            pl.BlockSpec((block_rows, k), lambda i: (i, 0)),
            pl.BlockSpec((k, n), lambda i: (0, 0)),
            pl.BlockSpec((1, n), lambda i: (0, 0)),
            pl.BlockSpec((1, n), lambda i: (0, 0)),
            pl.BlockSpec((1, n), lambda i: (0, 0)),
        ],
        out_specs=pl.BlockSpec((block_rows, n), lambda i: (i, 0)),
        out_shape=jax.ShapeDtypeStruct((m, n), jnp.float32),
    )(x, w, b.reshape(1, n), g.reshape(1, n), beta.reshape(1, n))


def _node_div_dense_ln_kernel(acc_ref, den_ref, w_ref, b_ref, g_ref, bb_ref,
                              o_ref):
    acc = acc_ref[...]
    den = den_ref[...]
    den_w = jnp.repeat(den, _OF, axis=1)
    h = jnp.where(den_w > 0, acc / den_w, 0.0)
    y = jnp.dot(h, w_ref[...], preferred_element_type=jnp.float32)
    y = jnp.maximum(y + b_ref[...], 0.0)
    mu = jnp.mean(y, axis=-1, keepdims=True)
    var = jnp.mean((y - mu) ** 2, axis=-1, keepdims=True)
    o_ref[...] = (y - mu) * lax.rsqrt(var + 1e-5) * g_ref[...] + bb_ref[...]


def _node_div_dense_ln(acc, den, w, b, g, beta, block_rows):
    m = acc.shape[0]
    n = w.shape[1]
    grid = (m // block_rows,)
    return pl.pallas_call(
        _node_div_dense_ln_kernel,
        grid=grid,
        in_specs=[
            pl.BlockSpec((block_rows, _HD), lambda i: (i, 0)),
            pl.BlockSpec((block_rows, _H), lambda i: (i, 0)),
            pl.BlockSpec((_HD, n), lambda i: (0, 0)),
            pl.BlockSpec((1, n), lambda i: (0, 0)),
            pl.BlockSpec((1, n), lambda i: (0, 0)),
            pl.BlockSpec((1, n), lambda i: (0, 0)),
        ],
        out_specs=pl.BlockSpec((block_rows, n), lambda i: (i, 0)),
        out_shape=jax.ShapeDtypeStruct((m, n), jnp.float32),
    )(acc, den, w, b.reshape(1, n), g.reshape(1, n), beta.reshape(1, n))


def kernel(node_feature, edge_feature, edge_index, W_ni, W_nj, W_fij, b_egat,
           attn, W_node_src, W_edge_dense, b_edge_dense, W_node_dense,
           b_node_dense, ln_gamma, ln_beta):
    src = edge_index[0]
    dst = edge_index[1]

    w3 = jnp.concatenate([W_ni, W_nj, W_node_src], axis=1)
    f_ni, f_nj, h_src = _proj(node_feature, w3, block_rows=1000)

    g = _sc_gather_add(f_ni, f_nj, src, dst)

    f_out, e = _edge_phase(edge_feature, W_fij, b_egat, g, attn.reshape(_HD),
                           block_rows=2000)

    kmax = jnp.max(e)
    ee = _exp_shift(e.reshape(_E * _H // 128, 128), kmax,
                    block_rows=5000).reshape(_E, _H)

    msg = h_src[src].reshape(_E, _H, _OF) * ee[:, :, None]
    acc = jax.ops.segment_sum(msg, dst, num_segments=_N).reshape(_N, _HD)
    den = jax.ops.segment_sum(ee, dst, num_segments=_N)

    edge_out = _dense_ln(f_out, W_edge_dense, b_edge_dense, ln_gamma, ln_beta,
                         block_rows=2000)
    node_out = _node_div_dense_ln(acc, den, W_node_dense, b_node_dense,
                                  ln_gamma, ln_beta, block_rows=1000)
    return (node_out, edge_out)


# speedup vs baseline: 6.3766x; 5.8316x over previous
import functools
import jax
import jax.numpy as jnp
from jax import lax
from jax.experimental import pallas as pl
from jax.experimental.pallas import tpu as pltpu
from jax.experimental.pallas import tpu_sc as plsc

_N = 10000
_E = 160000
_D = 256
_H = 4
_OF = 128
_HD = 512

_NW = 32
_EW = _E // _NW
_C = 40
_NCHUNK = _EW // _C


def _sc_mesh():
    return plsc.VectorSubcoreMesh(core_axis_name="c", subcore_axis_name="s",
                                  num_cores=2, num_subcores=16)


def _sc_gather_add_body(fni_hbm, fnj_hbm, src_hbm, dst_hbm, g_hbm,
                        src_v, dst_v, bufa, bufb, sema, semb):
    cid = lax.axis_index("c")
    sid = lax.axis_index("s")
    wid = sid * 2 + cid
    pltpu.sync_copy(src_hbm.at[wid], src_v)
    pltpu.sync_copy(dst_hbm.at[wid], dst_v)

    def chunk_body(c, carry):
        cpa = pltpu.async_copy(fni_hbm.at[src_v.at[c]], bufa, sema)
        cpb = pltpu.async_copy(fnj_hbm.at[dst_v.at[c]], bufb, semb)
        cpa.wait()
        cpb.wait()

        def row_body(r, carry2):
            def lane_body(l, carry3):
                sl = pl.ds(l * 16, 16)
                bufa[r, sl] = bufa[r, sl] + bufb[r, sl]
                return carry3
            return lax.fori_loop(0, _HD // 16, lane_body, carry2)
        lax.fori_loop(0, _C, row_body, carry)

        pltpu.sync_copy(bufa, g_hbm.at[pl.ds(wid * _EW + c * _C, _C)])
        return carry

    lax.fori_loop(0, _NCHUNK, chunk_body, 0)


def _sc_gather_add(f_ni, f_nj, src, dst):
    src2d = src.reshape(_NW, _NCHUNK, _C)
    dst2d = dst.reshape(_NW, _NCHUNK, _C)
    fn = pl.kernel(
        _sc_gather_add_body,
        out_type=jax.ShapeDtypeStruct((_E, _HD), jnp.float32),
        mesh=_sc_mesh(),
        scratch_types=[
            pltpu.VMEM((_NCHUNK, _C), jnp.int32),
            pltpu.VMEM((_NCHUNK, _C), jnp.int32),
            pltpu.VMEM((_C, _HD), jnp.float32),
            pltpu.VMEM((_C, _HD), jnp.float32),
            pltpu.SemaphoreType.DMA,
            pltpu.SemaphoreType.DMA,
        ],
    )
    return fn(f_ni, f_nj, src2d, dst2d)


_EW2 = _E // 16
_C2 = 80
_NCHUNK2 = _EW2 // _C2
_AW = 128
_NH2 = _N // 2


def _sc_message_body(h4_hbm, ee4_hbm, src4_hbm, dsth_hbm, out_hbm,
                     src_c, dst_c, ee_c, gbuf, zbuf, acc_sp, semg, semi):
    cid = lax.axis_index("c")
    sid = lax.axis_index("s")

    zeros16 = jnp.zeros((16,), jnp.float32)

    def zrow(r, carry):
        def zlane(k, c2):
            zbuf[r, pl.ds(k * 16, 16)] = zeros16
            return c2
        return lax.fori_loop(0, _AW // 16, zlane, carry)
    lax.fori_loop(0, 16, zrow, 0)

    for hh in range(2):
        h = cid * 2 + hh

        for half in range(2):
            def zseg(q, carry):
                pltpu.sync_copy(zbuf, acc_sp.at[pl.ds(sid * 312 + q * 16, 16)])
                return carry
            lax.fori_loop(0, 20, zseg, 0)
            plsc.subcore_barrier()

            def chunk_body(c, carry):
                ca = pltpu.async_copy(src4_hbm.at[h * 16 + sid, c], src_c,
                                      semi)
                cb = pltpu.async_copy(dsth_hbm.at[half * 16 + sid, c], dst_c,
                                      semi)
                cc = pltpu.async_copy(ee4_hbm.at[h * 16 + sid, c], ee_c, semi)
                ca.wait()
                cb.wait()
                cc.wait()
                pltpu.async_copy(h4_hbm.at[src_c.at[0]], gbuf, semg).wait()

                def group_body(gi, carry2):
                    vee = ee_c[gi, :]
                    e0 = gi * 16
                    for j in range(16):
                        s = vee[j]
                        for k in range(8):
                            sl = pl.ds(k * 16, 16)
                            gbuf[e0 + j, sl] = gbuf[e0 + j, sl] * s
                    return carry2
                lax.fori_loop(0, _C2 // 16, group_body, carry)

                pltpu.sync_copy(gbuf, acc_sp.at[dst_c.at[0]], add=True)
                return carry

            lax.fori_loop(0, _NCHUNK2, chunk_body, 0)
            plsc.subcore_barrier()

            pltpu.sync_copy(
                acc_sp.at[pl.ds(sid * 312, 312)],
                out_hbm.at[h, pl.ds(half * _NH2 + sid * 312, 312)])

            @pl.when(sid == 15)
            def _():
                pltpu.sync_copy(
                    acc_sp.at[pl.ds(4992, 8)],
                    out_hbm.at[h, pl.ds(half * _NH2 + 4992, 8)])
            plsc.subcore_barrier()


def _sc_message(h_src, ee, src, dst):
    h4 = h_src.reshape(_N * _H, _OF)
    ee4 = ee.T.reshape(_H * 16, _NCHUNK2, _C2 // 16, 16)
    src4 = (src * 4)[None, :] + jnp.arange(_H, dtype=jnp.int32)[:, None]
    src4 = src4.reshape(_H * 16, _NCHUNK2, 1, _C2)
    local = dst[None, :] - (jnp.arange(2, dtype=jnp.int32) * _NH2)[:, None]
    dsth = jnp.where((local >= 0) & (local < _NH2), local, _NH2)
    dsth = dsth.astype(jnp.int32).reshape(2 * 16, _NCHUNK2, 1, _C2)
    fn = pl.kernel(
        _sc_message_body,
        out_type=jax.ShapeDtypeStruct((_H, _N, _AW), jnp.float32),
        mesh=_sc_mesh(),
        scratch_types=[
            pltpu.VMEM((1, _C2), jnp.int32),
            pltpu.VMEM((1, _C2), jnp.int32),
            pltpu.VMEM((_C2 // 16, 16), jnp.float32),
            pltpu.VMEM((_C2, _OF), jnp.float32),
            pltpu.VMEM((16, _AW), jnp.float32),
            pltpu.VMEM_SHARED((_NH2 + 8, _AW), jnp.float32),
            pltpu.SemaphoreType.DMA,
            pltpu.SemaphoreType.DMA,
        ],
    )
    return fn(h4, ee4, src4, dsth)


def _proj_kernel(x_ref, w_ref, o1_ref, o2_ref, o3_ref):
    y = jnp.dot(x_ref[...], w_ref[...], preferred_element_type=jnp.float32)
    o1_ref[...] = y[:, :_HD]
    o2_ref[...] = y[:, _HD:2 * _HD]
    o3_ref[...] = y[:, 2 * _HD:]


def _proj(x, w3, block_rows):
    m, k = x.shape
    grid = (m // block_rows,)
    out = jax.ShapeDtypeStruct((m, _HD), jnp.float32)
    return pl.pallas_call(
        _proj_kernel,
        grid=grid,
        in_specs=[
            pl.BlockSpec((block_rows, k), lambda i: (i, 0)),
            pl.BlockSpec((k, 3 * _HD), lambda i: (0, 0)),
        ],
        out_specs=[pl.BlockSpec((block_rows, _HD), lambda i: (i, 0))] * 3,
        out_shape=[out, out, out],
    )(x, w3)


def _edge_phase_kernel(ef_ref, w_ref, b_ref, g_ref, attn_ref, fout_ref, e_ref):
    y = jnp.dot(ef_ref[...], w_ref[...], preferred_element_type=jnp.float32)
    f = y + b_ref[...] + g_ref[...]
    f = jnp.where(f > 0, f, 0.01 * f)
    fout_ref[...] = f
    fa = f * attn_ref[...]
    r = fa.shape[0]
    e_ref[...] = jnp.sum(fa.reshape(r, _H, _OF), axis=-1)


def _edge_phase(edge_feature, w_fij, b_egat, g, attn_flat, block_rows):
    grid = (_E // block_rows,)
    return pl.pallas_call(
        _edge_phase_kernel,
        grid=grid,
        in_specs=[
            pl.BlockSpec((block_rows, _D), lambda i: (i, 0)),
            pl.BlockSpec((_D, _HD), lambda i: (0, 0)),
            pl.BlockSpec((1, _HD), lambda i: (0, 0)),
            pl.BlockSpec((block_rows, _HD), lambda i: (i, 0)),
            pl.BlockSpec((1, _HD), lambda i: (0, 0)),
        ],
        out_specs=[
            pl.BlockSpec((block_rows, _HD), lambda i: (i, 0)),
            pl.BlockSpec((block_rows, _H), lambda i: (i, 0)),
        ],
        out_shape=[
            jax.ShapeDtypeStruct((_E, _HD), jnp.float32),
            jax.ShapeDtypeStruct((_E, _H), jnp.float32),
        ],
    )(edge_feature, w_fij, b_egat.reshape(1, _HD), g,
      attn_flat.reshape(1, _HD))


def _exp_kernel(e_ref, k_ref, o_ref):
    o_ref[...] = jnp.exp(e_ref[...] - k_ref[...])


def _exp_shift(e_flat, kmax, block_rows):
    m = e_flat.shape[0]
    grid = (m // block_rows,)
    return pl.pallas_call(
        _exp_kernel,
        grid=grid,
        in_specs=[
            pl.BlockSpec((block_rows, 128), lambda i: (i, 0)),
            pl.BlockSpec((1, 1), lambda i: (0, 0)),
        ],
        out_specs=pl.BlockSpec((block_rows, 128), lambda i: (i, 0)),
        out_shape=jax.ShapeDtypeStruct((m, 128), jnp.float32),
    )(e_flat, kmax.reshape(1, 1))


def _dense_ln_kernel(x_ref, w_ref, b_ref, g_ref, bb_ref, o_ref):
    y = jnp.dot(x_ref[...], w_ref[...], preferred_element_type=jnp.float32)
    y = jnp.maximum(y + b_ref[...], 0.0)
    mu = jnp.mean(y, axis=-1, keepdims=True)
    var = jnp.mean((y - mu) ** 2, axis=-1, keepdims=True)
    o_ref[...] = (y - mu) * lax.rsqrt(var + 1e-5) * g_ref[...] + bb_ref[...]


def _dense_ln(x, w, b, g, beta, block_rows):
    m, k = x.shape
    n = w.shape[1]
    grid = (m // block_rows,)
    return pl.pallas_call(
        _dense_ln_kernel,
        grid=grid,
        in_specs=[
            pl.BlockSpec((block_rows, k), lambda i: (i, 0)),
            pl.BlockSpec((k, n), lambda i: (0, 0)),
            pl.BlockSpec((1, n), lambda i: (0, 0)),
            pl.BlockSpec((1, n), lambda i: (0, 0)),
            pl.BlockSpec((1, n), lambda i: (0, 0)),
        ],
        out_specs=pl.BlockSpec((block_rows, n), lambda i: (i, 0)),
        out_shape=jax.ShapeDtypeStruct((m, n), jnp.float32),
    )(x, w, b.reshape(1, n), g.reshape(1, n), beta.reshape(1, n))


def _node_div_dense_ln_kernel(a0_ref, a1_ref, a2_ref, a3_ref, den_ref,
                              w_ref, b_ref, g_ref, bb_ref, o_ref):
    parts = []
    for hh, a_ref in enumerate((a0_ref, a1_ref, a2_ref, a3_ref)):
        acc = a_ref[0]
        den = den_ref[:, hh:hh + 1]
        parts.append(jnp.where(den > 0, acc / den, 0.0))
    h = jnp.concatenate(parts, axis=1)
    y = jnp.dot(h, w_ref[...], preferred_element_type=jnp.float32)
    y = jnp.maximum(y + b_ref[...], 0.0)
    mu = jnp.mean(y, axis=-1, keepdims=True)
    var = jnp.mean((y - mu) ** 2, axis=-1, keepdims=True)
    o_ref[...] = (y - mu) * lax.rsqrt(var + 1e-5) * g_ref[...] + bb_ref[...]


def _node_div_dense_ln(acc4, den, w, b, g, beta, block_rows):
    n = w.shape[1]
    grid = (_N // block_rows,)

    def plane_spec(h):
        return pl.BlockSpec((1, block_rows, _AW), lambda i, h=h: (h, i, 0))

    return pl.pallas_call(
        _node_div_dense_ln_kernel,
        grid=grid,
        in_specs=[
            plane_spec(0), plane_spec(1), plane_spec(2), plane_spec(3),
            pl.BlockSpec((block_rows, _H), lambda i: (i, 0)),
            pl.BlockSpec((_HD, n), lambda i: (0, 0)),
            pl.BlockSpec((1, n), lambda i: (0, 0)),
            pl.BlockSpec((1, n), lambda i: (0, 0)),
            pl.BlockSpec((1, n), lambda i: (0, 0)),
        ],
        out_specs=pl.BlockSpec((block_rows, n), lambda i: (i, 0)),
        out_shape=jax.ShapeDtypeStruct((_N, n), jnp.float32),
    )(acc4, acc4, acc4, acc4, den, w, b.reshape(1, n), g.reshape(1, n),
      beta.reshape(1, n))


def kernel(node_feature, edge_feature, edge_index, W_ni, W_nj, W_fij, b_egat,
           attn, W_node_src, W_edge_dense, b_edge_dense, W_node_dense,
           b_node_dense, ln_gamma, ln_beta):
    src = edge_index[0]
    dst = edge_index[1]

    w3 = jnp.concatenate([W_ni, W_nj, W_node_src], axis=1)
    f_ni, f_nj, h_src = _proj(node_feature, w3, block_rows=1000)

    g = _sc_gather_add(f_ni, f_nj, src, dst)

    f_out, e = _edge_phase(edge_feature, W_fij, b_egat, g, attn.reshape(_HD),
                           block_rows=2000)

    kmax = jnp.max(e)
    ee = _exp_shift(e.reshape(_E * _H // 128, 128), kmax,
                    block_rows=5000).reshape(_E, _H)

    acc4 = _sc_message(h_src, ee, src, dst)
    den = jax.ops.segment_sum(ee, dst, num_segments=_N)

    edge_out = _dense_ln(f_out, W_edge_dense, b_edge_dense, ln_gamma, ln_beta,
                         block_rows=2000)
    node_out = _node_div_dense_ln(acc4, den, W_node_dense, b_node_dense,
                                  ln_gamma, ln_beta, block_rows=1000)
    return (node_out, edge_out)


# speedup vs baseline: 7.6630x; 1.2018x over previous
import functools
import jax
import jax.numpy as jnp
from jax import lax
from jax.experimental import pallas as pl
from jax.experimental.pallas import tpu as pltpu
from jax.experimental.pallas import tpu_sc as plsc

_N = 10000
_E = 160000
_D = 256
_H = 4
_OF = 128
_HD = 512

_NW = 32
_EW = _E // _NW
_C = 40
_NCHUNK = _EW // _C


def _sc_mesh():
    return plsc.VectorSubcoreMesh(core_axis_name="c", subcore_axis_name="s",
                                  num_cores=2, num_subcores=16)


def _sc_gather_add_body(fni_hbm, fnj_hbm, src_hbm, dst_hbm, g_hbm,
                        src_v, dst_v, bufa, bufb, sema, semb):
    cid = lax.axis_index("c")
    sid = lax.axis_index("s")
    wid = sid * 2 + cid
    pltpu.sync_copy(src_hbm.at[wid], src_v)
    pltpu.sync_copy(dst_hbm.at[wid], dst_v)

    def chunk_body(c, carry):
        cpa = pltpu.async_copy(fni_hbm.at[src_v.at[c]], bufa, sema)
        cpb = pltpu.async_copy(fnj_hbm.at[dst_v.at[c]], bufb, semb)
        cpa.wait()
        cpb.wait()

        def row_body(r, carry2):
            for l in range(_HD // 16):
                sl = pl.ds(l * 16, 16)
                bufa[r, sl] = bufa[r, sl] + bufb[r, sl]
            return carry2
        lax.fori_loop(0, _C, row_body, carry)

        pltpu.sync_copy(bufa, g_hbm.at[pl.ds(wid * _EW + c * _C, _C)])
        return carry

    lax.fori_loop(0, _NCHUNK, chunk_body, 0)


def _sc_gather_add(f_ni, f_nj, src, dst):
    src2d = src.reshape(_NW, _NCHUNK, _C)
    dst2d = dst.reshape(_NW, _NCHUNK, _C)
    fn = pl.kernel(
        _sc_gather_add_body,
        out_type=jax.ShapeDtypeStruct((_E, _HD), jnp.float32),
        mesh=_sc_mesh(),
        scratch_types=[
            pltpu.VMEM((_NCHUNK, _C), jnp.int32),
            pltpu.VMEM((_NCHUNK, _C), jnp.int32),
            pltpu.VMEM((_C, _HD), jnp.float32),
            pltpu.VMEM((_C, _HD), jnp.float32),
            pltpu.SemaphoreType.DMA,
            pltpu.SemaphoreType.DMA,
        ],
    )
    return fn(f_ni, f_nj, src2d, dst2d)


_EW2 = _E // 16
_C2 = 80
_NCHUNK2 = _EW2 // _C2
_AW = 128
_NH2 = _N // 2


def _sc_message_body(h4_hbm, ee4_hbm, src4_hbm, dsth_hbm, out_hbm,
                     src_c, dst_c, ee_c, gbuf, zbuf, acc_sp, semg, semi):
    cid = lax.axis_index("c")
    sid = lax.axis_index("s")

    zeros16 = jnp.zeros((16,), jnp.float32)

    def zrow(r, carry):
        def zlane(k, c2):
            zbuf[r, pl.ds(k * 16, 16)] = zeros16
            return c2
        return lax.fori_loop(0, _AW // 16, zlane, carry)
    lax.fori_loop(0, 16, zrow, 0)

    for hh in range(2):
        h = cid * 2 + hh

        for half in range(2):
            def zseg(q, carry):
                pltpu.sync_copy(zbuf, acc_sp.at[pl.ds(sid * 312 + q * 16, 16)])
                return carry
            lax.fori_loop(0, 20, zseg, 0)
            plsc.subcore_barrier()

            def chunk_body(c, carry):
                ca = pltpu.async_copy(src4_hbm.at[h * 16 + sid, c], src_c,
                                      semi)
                cb = pltpu.async_copy(dsth_hbm.at[half * 16 + sid, c], dst_c,
                                      semi)
                cc = pltpu.async_copy(ee4_hbm.at[h * 16 + sid, c], ee_c, semi)
                ca.wait()
                cb.wait()
                cc.wait()
                pltpu.async_copy(h4_hbm.at[src_c.at[0]], gbuf, semg).wait()

                def group_body(gi, carry2):
                    vee = ee_c[gi, :]
                    e0 = gi * 16
                    for j in range(16):
                        s = vee[j]
                        for k in range(8):
                            sl = pl.ds(k * 16, 16)
                            gbuf[e0 + j, sl] = gbuf[e0 + j, sl] * s
                    return carry2
                lax.fori_loop(0, _C2 // 16, group_body, carry)

                pltpu.sync_copy(gbuf, acc_sp.at[dst_c.at[0]], add=True)
                return carry

            lax.fori_loop(0, _NCHUNK2, chunk_body, 0)
            plsc.subcore_barrier()

            pltpu.sync_copy(
                acc_sp.at[pl.ds(sid * 312, 312)],
                out_hbm.at[h, pl.ds(half * _NH2 + sid * 312, 312)])

            @pl.when(sid == 15)
            def _():
                pltpu.sync_copy(
                    acc_sp.at[pl.ds(4992, 8)],
                    out_hbm.at[h, pl.ds(half * _NH2 + 4992, 8)])
            plsc.subcore_barrier()


def _sc_message(h_src, ee, src, dst):
    h4 = h_src.reshape(_N * _H, _OF)
    ee4 = ee.T.reshape(_H * 16, _NCHUNK2, _C2 // 16, 16)
    src4 = (src * 4)[None, :] + jnp.arange(_H, dtype=jnp.int32)[:, None]
    src4 = src4.reshape(_H * 16, _NCHUNK2, 1, _C2)
    local = dst[None, :] - (jnp.arange(2, dtype=jnp.int32) * _NH2)[:, None]
    dsth = jnp.where((local >= 0) & (local < _NH2), local, _NH2)
    dsth = dsth.astype(jnp.int32).reshape(2 * 16, _NCHUNK2, 1, _C2)
    fn = pl.kernel(
        _sc_message_body,
        out_type=jax.ShapeDtypeStruct((_H, _N, _AW), jnp.float32),
        mesh=_sc_mesh(),
        scratch_types=[
            pltpu.VMEM((1, _C2), jnp.int32),
            pltpu.VMEM((1, _C2), jnp.int32),
            pltpu.VMEM((_C2 // 16, 16), jnp.float32),
            pltpu.VMEM((_C2, _OF), jnp.float32),
            pltpu.VMEM((16, _AW), jnp.float32),
            pltpu.VMEM_SHARED((_NH2 + 8, _AW), jnp.float32),
            pltpu.SemaphoreType.DMA,
            pltpu.SemaphoreType.DMA,
        ],
    )
    return fn(h4, ee4, src4, dsth)


def _proj_kernel(x_ref, w_ref, o1_ref, o2_ref, o3_ref):
    y = jnp.dot(x_ref[...], w_ref[...], preferred_element_type=jnp.float32)
    o1_ref[...] = y[:, :_HD]
    o2_ref[...] = y[:, _HD:2 * _HD]
    o3_ref[...] = y[:, 2 * _HD:]


def _proj(x, w3, block_rows):
    m, k = x.shape
    grid = (m // block_rows,)
    out = jax.ShapeDtypeStruct((m, _HD), jnp.float32)
    return pl.pallas_call(
        _proj_kernel,
        grid=grid,
        in_specs=[
            pl.BlockSpec((block_rows, k), lambda i: (i, 0)),
            pl.BlockSpec((k, 3 * _HD), lambda i: (0, 0)),
        ],
        out_specs=[pl.BlockSpec((block_rows, _HD), lambda i: (i, 0))] * 3,
        out_shape=[out, out, out],
    )(x, w3)


def _edge_phase_kernel(ef_ref, w_ref, b_ref, g_ref, attn_ref, fout_ref, e_ref):
    y = jnp.dot(ef_ref[...], w_ref[...], preferred_element_type=jnp.float32)
    f = y + b_ref[...] + g_ref[...]
    f = jnp.where(f > 0, f, 0.01 * f)
    fout_ref[...] = f
    fa = f * attn_ref[...]
    r = fa.shape[0]
    e_ref[...] = jnp.sum(fa.reshape(r, _H, _OF), axis=-1)


def _edge_phase(edge_feature, w_fij, b_egat, g, attn_flat, block_rows):
    grid = (_E // block_rows,)
    return pl.pallas_call(
        _edge_phase_kernel,
        grid=grid,
        in_specs=[
            pl.BlockSpec((block_rows, _D), lambda i: (i, 0)),
            pl.BlockSpec((_D, _HD), lambda i: (0, 0)),
            pl.BlockSpec((1, _HD), lambda i: (0, 0)),
            pl.BlockSpec((block_rows, _HD), lambda i: (i, 0)),
            pl.BlockSpec((1, _HD), lambda i: (0, 0)),
        ],
        out_specs=[
            pl.BlockSpec((block_rows, _HD), lambda i: (i, 0)),
            pl.BlockSpec((block_rows, _H), lambda i: (i, 0)),
        ],
        out_shape=[
            jax.ShapeDtypeStruct((_E, _HD), jnp.float32),
            jax.ShapeDtypeStruct((_E, _H), jnp.float32),
        ],
    )(edge_feature, w_fij, b_egat.reshape(1, _HD), g,
      attn_flat.reshape(1, _HD))


def _exp_kernel(e_ref, k_ref, o_ref):
    o_ref[...] = jnp.exp(e_ref[...] - k_ref[...])


def _exp_shift(e_flat, kmax, block_rows):
    m = e_flat.shape[0]
    grid = (m // block_rows,)
    return pl.pallas_call(
        _exp_kernel,
        grid=grid,
        in_specs=[
            pl.BlockSpec((block_rows, 128), lambda i: (i, 0)),
            pl.BlockSpec((1, 1), lambda i: (0, 0)),
        ],
        out_specs=pl.BlockSpec((block_rows, 128), lambda i: (i, 0)),
        out_shape=jax.ShapeDtypeStruct((m, 128), jnp.float32),
    )(e_flat, kmax.reshape(1, 1))


def _dense_ln_kernel(x_ref, w_ref, b_ref, g_ref, bb_ref, o_ref):
    y = jnp.dot(x_ref[...], w_ref[...], preferred_element_type=jnp.float32)
    y = jnp.maximum(y + b_ref[...], 0.0)
    mu = jnp.mean(y, axis=-1, keepdims=True)
    var = jnp.mean((y - mu) ** 2, axis=-1, keepdims=True)
    o_ref[...] = (y - mu) * lax.rsqrt(var + 1e-5) * g_ref[...] + bb_ref[...]


def _dense_ln(x, w, b, g, beta, block_rows):
    m, k = x.shape
    n = w.shape[1]
    grid = (m // block_rows,)
    return pl.pallas_call(
        _dense_ln_kernel,
        grid=grid,
        in_specs=[
            pl.BlockSpec((block_rows, k), lambda i: (i, 0)),
            pl.BlockSpec((k, n), lambda i: (0, 0)),
            pl.BlockSpec((1, n), lambda i: (0, 0)),
            pl.BlockSpec((1, n), lambda i: (0, 0)),
            pl.BlockSpec((1, n), lambda i: (0, 0)),
        ],
        out_specs=pl.BlockSpec((block_rows, n), lambda i: (i, 0)),
        out_shape=jax.ShapeDtypeStruct((m, n), jnp.float32),
    )(x, w, b.reshape(1, n), g.reshape(1, n), beta.reshape(1, n))


def _node_div_dense_ln_kernel(a0_ref, a1_ref, a2_ref, a3_ref, den_ref,
                              w_ref, b_ref, g_ref, bb_ref, o_ref):
    parts = []
    for hh, a_ref in enumerate((a0_ref, a1_ref, a2_ref, a3_ref)):
        acc = a_ref[0]
        den = den_ref[:, hh:hh + 1]
        parts.append(jnp.where(den > 0, acc / den, 0.0))
    h = jnp.concatenate(parts, axis=1)
    y = jnp.dot(h, w_ref[...], preferred_element_type=jnp.float32)
    y = jnp.maximum(y + b_ref[...], 0.0)
    mu = jnp.mean(y, axis=-1, keepdims=True)
    var = jnp.mean((y - mu) ** 2, axis=-1, keepdims=True)
    o_ref[...] = (y - mu) * lax.rsqrt(var + 1e-5) * g_ref[...] + bb_ref[...]


def _node_div_dense_ln(acc4, den, w, b, g, beta, block_rows):
    n = w.shape[1]
    grid = (_N // block_rows,)

    def plane_spec(h):
        return pl.BlockSpec((1, block_rows, _AW), lambda i, h=h: (h, i, 0))

    return pl.pallas_call(
        _node_div_dense_ln_kernel,
        grid=grid,
        in_specs=[
            plane_spec(0), plane_spec(1), plane_spec(2), plane_spec(3),
            pl.BlockSpec((block_rows, _H), lambda i: (i, 0)),
            pl.BlockSpec((_HD, n), lambda i: (0, 0)),
            pl.BlockSpec((1, n), lambda i: (0, 0)),
            pl.BlockSpec((1, n), lambda i: (0, 0)),
            pl.BlockSpec((1, n), lambda i: (0, 0)),
        ],
        out_specs=pl.BlockSpec((block_rows, n), lambda i: (i, 0)),
        out_shape=jax.ShapeDtypeStruct((_N, n), jnp.float32),
    )(acc4, acc4, acc4, acc4, den, w, b.reshape(1, n), g.reshape(1, n),
      beta.reshape(1, n))


def kernel(node_feature, edge_feature, edge_index, W_ni, W_nj, W_fij, b_egat,
           attn, W_node_src, W_edge_dense, b_edge_dense, W_node_dense,
           b_node_dense, ln_gamma, ln_beta):
    src = edge_index[0]
    dst = edge_index[1]

    w3 = jnp.concatenate([W_ni, W_nj, W_node_src], axis=1)
    f_ni, f_nj, h_src = _proj(node_feature, w3, block_rows=1000)

    g = _sc_gather_add(f_ni, f_nj, src, dst)

    f_out, e = _edge_phase(edge_feature, W_fij, b_egat, g, attn.reshape(_HD),
                           block_rows=2000)

    kmax = jnp.max(e)
    ee = _exp_shift(e.reshape(_E * _H // 128, 128), kmax,
                    block_rows=5000).reshape(_E, _H)

    acc4 = _sc_message(h_src, ee, src, dst)
    den = jax.ops.segment_sum(ee, dst, num_segments=_N)

    edge_out = _dense_ln(f_out, W_edge_dense, b_edge_dense, ln_gamma, ln_beta,
                         block_rows=2000)
    node_out = _node_div_dense_ln(acc4, den, W_node_dense, b_node_dense,
                                  ln_gamma, ln_beta, block_rows=1000)
    return (node_out, edge_out)
